# baseline (speedup 1.0000x reference)
import jax
import jax.numpy as jnp
from jax.experimental import pallas as pl
from jax.experimental.pallas import tpu as pltpu

_B, _T, _RANK = 8, 30, 8
_H = _W = 64
_RH = _RW = 14
_N = _T * _RANK


def _lrelu(v):
    return jnp.where(v >= 0, v, 0.01 * v)


def _pad_hw3(v, p):
    n, h, w = v.shape
    z = jnp.zeros((n, h, p), v.dtype)
    v = jnp.concatenate([z, v, z], axis=2)
    z = jnp.zeros((n, p, w + 2 * p), v.dtype)
    return jnp.concatenate([z, v, z], axis=1)


def _pad_hw4(v, p):
    n, h, w, c = v.shape
    z = jnp.zeros((n, h, p, c), v.dtype)
    v = jnp.concatenate([z, v, z], axis=2)
    z = jnp.zeros((n, p, w + 2 * p, c), v.dtype)
    return jnp.concatenate([z, v, z], axis=1)


def _conv5x5_c1(v, w1):
    n, h, w = v.shape
    vp = _pad_hw3(v, 2)
    acc = jnp.zeros((n, h, w, 32), jnp.float32)
    for dy in range(5):
        for dx in range(5):
            acc = acc + vp[:, dy:dy + h, dx:dx + w][..., None] * w1[dy * 5 + dx]
    return acc


def _conv3x3(v, wr, cout):
    n, h, w, cin = v.shape
    vp = _pad_hw4(v, 1)
    taps = [vp[:, dy:dy + h, dx:dx + w, :] for dy in range(3) for dx in range(3)]
    im = jnp.concatenate(taps, axis=-1)
    out = jnp.dot(im.reshape(n * h * w, 9 * cin), wr,
                  preferred_element_type=jnp.float32)
    return out.reshape(n, h, w, cout)


def _pool2(v):
    n, h, w, c = v.shape
    h2, w2 = h // 2, w // 2
    v = v[:, :2 * h2, :2 * w2, :]
    v = jnp.max(v.reshape(n, h2, 2, 2 * w2, c), axis=2)
    v = jnp.max(v.reshape(n, h2, w2, 2, c), axis=3)
    return v


def _base_cl(v, w1, w2, w3, w4, w5, w6):
    v = _lrelu(_conv5x5_c1(v, w1))
    v = _pool2(_lrelu(_conv3x3(v, w2, 32)))
    v = _lrelu(_conv3x3(v, w3, 64))
    v = _pool2(_lrelu(_conv3x3(v, w4, 64)))
    v = _lrelu(_conv3x3(v, w5, 128))
    v = _lrelu(_conv3x3(v, w6, 128))
    return v


def _global_body(x_ref, w1_ref, w2_ref, w3_ref, w4_ref, w5_ref, w6_ref, o_ref):
    t = pl.program_id(1)
    v = _base_cl(x_ref[0], w1_ref[...], w2_ref[...], w3_ref[...],
                 w4_ref[...], w5_ref[...], w6_ref[...])
    f = v.reshape(256, 128)

    @pl.when(t == 0)
    def _():
        o_ref[0] = f

    @pl.when(t != 0)
    def _():
        o_ref[0] = jnp.maximum(o_ref[0], f)


def _run_global(x, w1, w2, w3, w4, w5, w6):
    wspec = [pl.BlockSpec(w.shape, lambda b, t: (0, 0))
             for w in (w1, w2, w3, w4, w5, w6)]
    return pl.pallas_call(
        _global_body,
        out_shape=jax.ShapeDtypeStruct((_B, 256, 128), jnp.float32),
        grid=(_B, _T),
        in_specs=[pl.BlockSpec((1, 1, _H, _W), lambda b, t: (b, t, 0, 0))] + wspec,
        out_specs=pl.BlockSpec((1, 256, 128), lambda b, t: (b, 0, 0)),
        compiler_params=pltpu.CompilerParams(
            dimension_semantics=("parallel", "arbitrary"),
            vmem_limit_bytes=100 * 1024 * 1024,
        ),
        name="rgp_global",
    )(x, w1, w2, w3, w4, w5, w6)


def _local_body(x_ref, px_ref, w1_ref, w2_ref, w3_ref, w4_ref, w5_ref, w6_ref,
                fcw_ref, fcb_ref, o_ref):
    img = x_ref[0, 0]
    pxv = px_ref[0, 0]
    cxc = pxv[:, 0:1]
    cyc = pxv[:, 1:2]
    x1 = jnp.floor(jnp.clip(cxc - 7.0, 0.0, _W - 1.0))
    x2 = jnp.floor(jnp.minimum(cxc + 7.0, float(_W)))
    y1 = jnp.floor(jnp.clip(cyc - 7.0, 0.0, _H - 1.0))
    y2 = jnp.floor(jnp.minimum(cyc + 7.0, float(_H)))
    u = jax.lax.broadcasted_iota(jnp.int32, (_RANK, _RW), 1).astype(jnp.float32) + 0.5
    sx = jnp.clip(x1 + u * (x2 - x1) / _RW - 0.5, x1, x2 - 1.0)
    sy = jnp.clip(y1 + u * (y2 - y1) / _RH - 0.5, y1, y2 - 1.0)
    ix0 = jnp.floor(sx)
    wx = sx - ix0
    ix1 = jnp.minimum(ix0 + 1.0, x2 - 1.0)
    iy0 = jnp.floor(sy)
    wy = sy - iy0
    iy1 = jnp.minimum(iy0 + 1.0, y2 - 1.0)
    lane = jax.lax.broadcasted_iota(jnp.int32, (_RANK, _RW, _W), 2).astype(jnp.float32)
    rx = ((lane == ix0[..., None]) * (1.0 - wx)[..., None]
          + (lane == ix1[..., None]) * wx[..., None])
    ry = ((lane == iy0[..., None]) * (1.0 - wy)[..., None]
          + (lane == iy1[..., None]) * wy[..., None])
    rows = []
    for r in range(_RANK):
        t1 = jnp.dot(ry[r], img, preferred_element_type=jnp.float32)
        c = jax.lax.dot_general(t1, rx[r], (((1,), (1,)), ((), ())),
                                preferred_element_type=jnp.float32)
        rows.append(c[None])
    v = jnp.concatenate(rows, axis=0)
    v = _base_cl(v, w1_ref[...], w2_ref[...], w3_ref[...],
                 w4_ref[...], w5_ref[...], w6_ref[...])
    acc = jnp.zeros((_RANK, 256), jnp.float32) + fcb_ref[...]
    for i in range(3):
        for j in range(3):
            acc = acc + jnp.dot(v[:, i, j, :], fcw_ref[i, j],
                                preferred_element_type=jnp.float32)
    o_ref[0, 0] = acc


def _run_local(x, px, w1, w2, w3, w4, w5, w6, fcw, fcb):
    wspec = [pl.BlockSpec(w.shape, lambda b, t: (0, 0))
             for w in (w1, w2, w3, w4, w5, w6)]
    return pl.pallas_call(
        _local_body,
        out_shape=jax.ShapeDtypeStruct((_B, _T, _RANK, 256), jnp.float32),
        grid=(_B, _T),
        in_specs=[pl.BlockSpec((1, 1, _H, _W), lambda b, t: (b, t, 0, 0)),
                  pl.BlockSpec((1, 1, _RANK, 2), lambda b, t: (b, t, 0, 0))]
                 + wspec
                 + [pl.BlockSpec((3, 3, 128, 256), lambda b, t: (0, 0, 0, 0)),
                    pl.BlockSpec((1, 256), lambda b, t: (0, 0))],
        out_specs=pl.BlockSpec((1, 1, _RANK, 256), lambda b, t: (b, t, 0, 0)),
        compiler_params=pltpu.CompilerParams(
            dimension_semantics=("parallel", "arbitrary"),
            vmem_limit_bytes=100 * 1024 * 1024,
        ),
        name="rgp_local",
    )(x, px, w1, w2, w3, w4, w5, w6, fcw, fcb)


_KCH = 2048


def _fcg_body(g_ref, w_ref, b_ref, o_ref):
    k = pl.program_id(0)

    @pl.when(k == 0)
    def _():
        o_ref[...] = jnp.zeros_like(o_ref) + b_ref[...]

    o_ref[...] += jnp.dot(g_ref[...], w_ref[...],
                          preferred_element_type=jnp.float32)


def _run_fcg(g, w, b):
    nk = g.shape[1] // _KCH
    return pl.pallas_call(
        _fcg_body,
        out_shape=jax.ShapeDtypeStruct((_B, 256), jnp.float32),
        grid=(nk,),
        in_specs=[pl.BlockSpec((_B, _KCH), lambda k: (0, k)),
                  pl.BlockSpec((_KCH, 256), lambda k: (k, 0)),
                  pl.BlockSpec((1, 256), lambda k: (0, 0))],
        out_specs=pl.BlockSpec((_B, 256), lambda k: (0, 0)),
        compiler_params=pltpu.CompilerParams(
            dimension_semantics=("arbitrary",),
        ),
        name="rgp_fcg",
    )(g, w, b)


def _gat_body(lfc_ref, gfc_ref, w1_ref, a11_ref, a12_ref,
              w2_ref, a21_ref, a22_ref, o_ref):
    hin = lfc_ref[0]
    ii = jax.lax.broadcasted_iota(jnp.int32, (_N, _N), 0)
    jj = jax.lax.broadcasted_iota(jnp.int32, (_N, _N), 1)
    fi = ii // _RANK
    fj = jj // _RANK
    adj = (fi == fj) | (((ii % _RANK) == (jj % _RANK)) & (jnp.abs(fi - fj) == 1))

    def layer(h_in, wm, a1, a2):
        h = jnp.dot(h_in, wm, preferred_element_type=jnp.float32)
        s1 = jnp.dot(h, a1, preferred_element_type=jnp.float32)
        s2 = jnp.dot(h, a2, preferred_element_type=jnp.float32)
        e = s1 + jnp.transpose(s2)
        e = jnp.where(e >= 0, e, 0.2 * e)
        e = jnp.where(adj, e, -9e15)
        m = jnp.max(e, axis=-1, keepdims=True)
        p = jnp.exp(e - m)
        attn = p / jnp.sum(p, axis=-1, keepdims=True)
        return jnp.dot(attn, h, preferred_element_type=jnp.float32)

    h1 = layer(hin, w1_ref[...], a11_ref[...], a12_ref[...])
    h2 = layer(h1, w2_ref[...], a21_ref[...], a22_ref[...])
    g2 = jnp.where(h2 > 0, h2, jnp.exp(h2) - 1.0)
    mg = jnp.max(g2, axis=-1, keepdims=True)
    lse = jnp.log(jnp.sum(jnp.exp(g2 - mg), axis=-1, keepdims=True))
    gat_fc = g2 - mg - lse
    meanl = jnp.mean(hin, axis=0)
    meang = jnp.mean(gat_fc, axis=0)
    fc = jnp.concatenate([gfc_ref[0, 0], meanl, meang])
    feats = [fc * 2.0]
    for nb in (2, 4, 8, 16):
        c = 528 // nb
        s = fc[0:c]
        mx = fc[0:c]
        for i in range(1, nb):
            ch = fc[i * c:(i + 1) * c]
            s = s + ch
            mx = jnp.maximum(mx, ch)
        feats.append(s * (1.0 / nb) + mx)
    o_ref[0, 0] = jnp.concatenate(feats)


def _run_gat(lfc, gfc, w1, a11, a12, w2, a21, a22):
    return pl.pallas_call(
        _gat_body,
        out_shape=jax.ShapeDtypeStruct((_B, 1, 1023), jnp.float32),
        grid=(_B,),
        in_specs=[pl.BlockSpec((1, _N, 256), lambda b: (b, 0, 0)),
                  pl.BlockSpec((1, 1, 256), lambda b: (b, 0, 0)),
                  pl.BlockSpec((256, 8), lambda b: (0, 0)),
                  pl.BlockSpec((8, 1), lambda b: (0, 0)),
                  pl.BlockSpec((8, 1), lambda b: (0, 0)),
                  pl.BlockSpec((8, 16), lambda b: (0, 0)),
                  pl.BlockSpec((16, 1), lambda b: (0, 0)),
                  pl.BlockSpec((16, 1), lambda b: (0, 0))],
        out_specs=pl.BlockSpec((1, 1, 1023), lambda b: (b, 0, 0)),
        compiler_params=pltpu.CompilerParams(
            dimension_semantics=("parallel",),
        ),
        name="rgp_gat",
    )(lfc, gfc, w1, a11, a12, w2, a21, a22)


def kernel(x, px, conv1, conv2, conv3, conv4, conv5, conv6,
           fc_g_w, fc_g_b, fc_l_w, fc_l_b,
           gat1_W, gat1_a1, gat1_a2, gat2_W, gat2_a1, gat2_a2):
    w1 = conv1.transpose(2, 3, 1, 0).reshape(25, 32)
    w2 = conv2.transpose(2, 3, 1, 0).reshape(288, 32)
    w3 = conv3.transpose(2, 3, 1, 0).reshape(288, 64)
    w4 = conv4.transpose(2, 3, 1, 0).reshape(576, 64)
    w5 = conv5.transpose(2, 3, 1, 0).reshape(576, 128)
    w6 = conv6.transpose(2, 3, 1, 0).reshape(1152, 128)

    g = _run_global(x, w1, w2, w3, w4, w5, w6)
    wgp = fc_g_w.reshape(256, 128, 16, 16).transpose(2, 3, 1, 0).reshape(32768, 256)
    g_fc = _run_fcg(g.reshape(_B, 256 * 128), wgp, fc_g_b.reshape(1, 256))

    fcwp = fc_l_w.reshape(256, 128, 3, 3).transpose(2, 3, 1, 0)
    lfc4 = _run_local(x, px, w1, w2, w3, w4, w5, w6, fcwp, fc_l_b.reshape(1, 256))
    lfc = lfc4.reshape(_B, _N, 256)

    return _run_gat(lfc, g_fc.reshape(_B, 1, 256),
                    gat1_W, gat1_a1, gat1_a2, gat2_W, gat2_a1, gat2_a2)


# speedup vs baseline: 1.0104x; 1.0104x over previous
import jax
import jax.numpy as jnp
from jax.experimental import pallas as pl
from jax.experimental.pallas import tpu as pltpu

_B, _T, _RANK = 8, 30, 8
_H = _W = 64
_RH = _RW = 14
_N = _T * _RANK


def _lrelu(v):
    return jnp.where(v >= 0, v, 0.01 * v)


def _pad_hw3(v, p):
    n, h, w = v.shape
    z = jnp.zeros((n, h, p), v.dtype)
    v = jnp.concatenate([z, v, z], axis=2)
    z = jnp.zeros((n, p, w + 2 * p), v.dtype)
    return jnp.concatenate([z, v, z], axis=1)


def _pad_hw4(v, p):
    n, h, w, c = v.shape
    z = jnp.zeros((n, h, p, c), v.dtype)
    v = jnp.concatenate([z, v, z], axis=2)
    z = jnp.zeros((n, p, w + 2 * p, c), v.dtype)
    return jnp.concatenate([z, v, z], axis=1)


def _conv5x5_c1(v, w1):
    n, h, w = v.shape
    vp = _pad_hw3(v, 2)
    acc = jnp.zeros((n, h, w, 32), jnp.float32)
    for dy in range(5):
        for dx in range(5):
            acc = acc + vp[:, dy:dy + h, dx:dx + w][..., None] * w1[dy * 5 + dx]
    return acc


def _conv3x3(v, wr, cout):
    n, h, w, cin = v.shape
    vp = _pad_hw4(v, 1)
    taps = [vp[:, dy:dy + h, dx:dx + w, :] for dy in range(3) for dx in range(3)]
    im = jnp.concatenate(taps, axis=-1)
    out = jnp.dot(im.reshape(n * h * w, 9 * cin), wr,
                  preferred_element_type=jnp.float32)
    return out.reshape(n, h, w, cout)


def _pool2(v):
    n, h, w, c = v.shape
    h2, w2 = h // 2, w // 2
    v = v[:, :2 * h2, :2 * w2, :]
    v = jnp.max(v.reshape(n, h2, 2, 2 * w2, c), axis=2)
    v = jnp.max(v.reshape(n, h2, w2, 2, c), axis=3)
    return v


def _base_cl(v, w1, w2, w3, w4, w5, w6):
    v = _lrelu(_conv5x5_c1(v, w1))
    v = _pool2(_lrelu(_conv3x3(v, w2, 32)))
    v = _lrelu(_conv3x3(v, w3, 64))
    v = _pool2(_lrelu(_conv3x3(v, w4, 64)))
    v = _lrelu(_conv3x3(v, w5, 128))
    v = _lrelu(_conv3x3(v, w6, 128))
    return v


def _global_body(x_ref, w1_ref, w2_ref, w3_ref, w4_ref, w5_ref, w6_ref, o_ref):
    t = pl.program_id(1)
    v = _base_cl(x_ref[0], w1_ref[...], w2_ref[...], w3_ref[...],
                 w4_ref[...], w5_ref[...], w6_ref[...])
    f = v.reshape(256, 128)

    @pl.when(t == 0)
    def _():
        o_ref[0] = f

    @pl.when(t != 0)
    def _():
        o_ref[0] = jnp.maximum(o_ref[0], f)


def _run_global(x, w1, w2, w3, w4, w5, w6):
    wspec = [pl.BlockSpec(w.shape, lambda b, t: (0, 0))
             for w in (w1, w2, w3, w4, w5, w6)]
    return pl.pallas_call(
        _global_body,
        out_shape=jax.ShapeDtypeStruct((_B, 256, 128), jnp.float32),
        grid=(_B, _T),
        in_specs=[pl.BlockSpec((1, 1, _H, _W), lambda b, t: (b, t, 0, 0))] + wspec,
        out_specs=pl.BlockSpec((1, 256, 128), lambda b, t: (b, 0, 0)),
        compiler_params=pltpu.CompilerParams(
            dimension_semantics=("parallel", "arbitrary"),
            vmem_limit_bytes=100 * 1024 * 1024,
        ),
        name="rgp_global",
    )(x, w1, w2, w3, w4, w5, w6)


def _local_body(x_ref, px_ref, w1_ref, w2_ref, w3_ref, w4_ref, w5_ref, w6_ref,
                fcw_ref, fcb_ref, o_ref):
    img = x_ref[0, 0]
    pxv = px_ref[0, 0]
    cxc = pxv[:, 0:1]
    cyc = pxv[:, 1:2]
    x1 = jnp.floor(jnp.clip(cxc - 7.0, 0.0, _W - 1.0))
    x2 = jnp.floor(jnp.minimum(cxc + 7.0, float(_W)))
    y1 = jnp.floor(jnp.clip(cyc - 7.0, 0.0, _H - 1.0))
    y2 = jnp.floor(jnp.minimum(cyc + 7.0, float(_H)))
    u = jax.lax.broadcasted_iota(jnp.int32, (_RANK, _RW), 1).astype(jnp.float32) + 0.5
    sx = jnp.clip(x1 + u * (x2 - x1) / _RW - 0.5, x1, x2 - 1.0)
    sy = jnp.clip(y1 + u * (y2 - y1) / _RH - 0.5, y1, y2 - 1.0)
    ix0 = jnp.floor(sx)
    wx = sx - ix0
    ix1 = jnp.minimum(ix0 + 1.0, x2 - 1.0)
    iy0 = jnp.floor(sy)
    wy = sy - iy0
    iy1 = jnp.minimum(iy0 + 1.0, y2 - 1.0)
    lane = jax.lax.broadcasted_iota(jnp.int32, (_RANK, _RW, _W), 2).astype(jnp.float32)
    rx = ((lane == ix0[..., None]) * (1.0 - wx)[..., None]
          + (lane == ix1[..., None]) * wx[..., None])
    ry = ((lane == iy0[..., None]) * (1.0 - wy)[..., None]
          + (lane == iy1[..., None]) * wy[..., None])
    rya = ry.reshape(_RANK * _RH, _W)
    rxa = rx.reshape(_RANK * _RW, _W)
    t1 = jnp.dot(rya, img, preferred_element_type=jnp.float32)
    cc = jax.lax.dot_general(t1, rxa, (((1,), (1,)), ((), ())),
                             preferred_element_type=jnp.float32)
    rows = [cc[r * _RH:(r + 1) * _RH, r * _RW:(r + 1) * _RW][None]
            for r in range(_RANK)]
    v = jnp.concatenate(rows, axis=0)
    v = _base_cl(v, w1_ref[...], w2_ref[...], w3_ref[...],
                 w4_ref[...], w5_ref[...], w6_ref[...])
    acc = jnp.zeros((_RANK, 256), jnp.float32) + fcb_ref[...]
    for i in range(3):
        for j in range(3):
            acc = acc + jnp.dot(v[:, i, j, :], fcw_ref[i, j],
                                preferred_element_type=jnp.float32)
    o_ref[0, 0] = acc


def _run_local(x, px, w1, w2, w3, w4, w5, w6, fcw, fcb):
    wspec = [pl.BlockSpec(w.shape, lambda b, t: (0, 0))
             for w in (w1, w2, w3, w4, w5, w6)]
    return pl.pallas_call(
        _local_body,
        out_shape=jax.ShapeDtypeStruct((_B, _T, _RANK, 256), jnp.float32),
        grid=(_B, _T),
        in_specs=[pl.BlockSpec((1, 1, _H, _W), lambda b, t: (b, t, 0, 0)),
                  pl.BlockSpec((1, 1, _RANK, 2), lambda b, t: (b, t, 0, 0))]
                 + wspec
                 + [pl.BlockSpec((3, 3, 128, 256), lambda b, t: (0, 0, 0, 0)),
                    pl.BlockSpec((1, 256), lambda b, t: (0, 0))],
        out_specs=pl.BlockSpec((1, 1, _RANK, 256), lambda b, t: (b, t, 0, 0)),
        compiler_params=pltpu.CompilerParams(
            dimension_semantics=("parallel", "arbitrary"),
            vmem_limit_bytes=100 * 1024 * 1024,
        ),
        name="rgp_local",
    )(x, px, w1, w2, w3, w4, w5, w6, fcw, fcb)


_KCH = 2048


def _fcg_body(g_ref, w_ref, b_ref, o_ref):
    k = pl.program_id(0)

    @pl.when(k == 0)
    def _():
        o_ref[...] = jnp.zeros_like(o_ref) + b_ref[...]

    o_ref[...] += jnp.dot(g_ref[...], w_ref[...],
                          preferred_element_type=jnp.float32)


def _run_fcg(g, w, b):
    nk = g.shape[1] // _KCH
    return pl.pallas_call(
        _fcg_body,
        out_shape=jax.ShapeDtypeStruct((_B, 256), jnp.float32),
        grid=(nk,),
        in_specs=[pl.BlockSpec((_B, _KCH), lambda k: (0, k)),
                  pl.BlockSpec((_KCH, 256), lambda k: (k, 0)),
                  pl.BlockSpec((1, 256), lambda k: (0, 0))],
        out_specs=pl.BlockSpec((_B, 256), lambda k: (0, 0)),
        compiler_params=pltpu.CompilerParams(
            dimension_semantics=("arbitrary",),
        ),
        name="rgp_fcg",
    )(g, w, b)


def _gat_body(lfc_ref, gfc_ref, w1_ref, a11_ref, a12_ref,
              w2_ref, a21_ref, a22_ref, o_ref):
    hin = lfc_ref[0]
    ii = jax.lax.broadcasted_iota(jnp.int32, (_N, _N), 0)
    jj = jax.lax.broadcasted_iota(jnp.int32, (_N, _N), 1)
    fi = ii // _RANK
    fj = jj // _RANK
    adj = (fi == fj) | (((ii % _RANK) == (jj % _RANK)) & (jnp.abs(fi - fj) == 1))

    def layer(h_in, wm, a1, a2):
        h = jnp.dot(h_in, wm, preferred_element_type=jnp.float32)
        s1 = jnp.dot(h, a1, preferred_element_type=jnp.float32)
        s2 = jnp.dot(h, a2, preferred_element_type=jnp.float32)
        e = s1 + jnp.transpose(s2)
        e = jnp.where(e >= 0, e, 0.2 * e)
        e = jnp.where(adj, e, -9e15)
        m = jnp.max(e, axis=-1, keepdims=True)
        p = jnp.exp(e - m)
        attn = p / jnp.sum(p, axis=-1, keepdims=True)
        return jnp.dot(attn, h, preferred_element_type=jnp.float32)

    h1 = layer(hin, w1_ref[...], a11_ref[...], a12_ref[...])
    h2 = layer(h1, w2_ref[...], a21_ref[...], a22_ref[...])
    g2 = jnp.where(h2 > 0, h2, jnp.exp(h2) - 1.0)
    mg = jnp.max(g2, axis=-1, keepdims=True)
    lse = jnp.log(jnp.sum(jnp.exp(g2 - mg), axis=-1, keepdims=True))
    gat_fc = g2 - mg - lse
    meanl = jnp.mean(hin, axis=0)
    meang = jnp.mean(gat_fc, axis=0)
    fc = jnp.concatenate([gfc_ref[0, 0], meanl, meang])
    feats = [fc * 2.0]
    for nb in (2, 4, 8, 16):
        c = 528 // nb
        s = fc[0:c]
        mx = fc[0:c]
        for i in range(1, nb):
            ch = fc[i * c:(i + 1) * c]
            s = s + ch
            mx = jnp.maximum(mx, ch)
        feats.append(s * (1.0 / nb) + mx)
    o_ref[0, 0] = jnp.concatenate(feats)


def _run_gat(lfc, gfc, w1, a11, a12, w2, a21, a22):
    return pl.pallas_call(
        _gat_body,
        out_shape=jax.ShapeDtypeStruct((_B, 1, 1023), jnp.float32),
        grid=(_B,),
        in_specs=[pl.BlockSpec((1, _N, 256), lambda b: (b, 0, 0)),
                  pl.BlockSpec((1, 1, 256), lambda b: (b, 0, 0)),
                  pl.BlockSpec((256, 8), lambda b: (0, 0)),
                  pl.BlockSpec((8, 1), lambda b: (0, 0)),
                  pl.BlockSpec((8, 1), lambda b: (0, 0)),
                  pl.BlockSpec((8, 16), lambda b: (0, 0)),
                  pl.BlockSpec((16, 1), lambda b: (0, 0)),
                  pl.BlockSpec((16, 1), lambda b: (0, 0))],
        out_specs=pl.BlockSpec((1, 1, 1023), lambda b: (b, 0, 0)),
        compiler_params=pltpu.CompilerParams(
            dimension_semantics=("parallel",),
        ),
        name="rgp_gat",
    )(lfc, gfc, w1, a11, a12, w2, a21, a22)


def kernel(x, px, conv1, conv2, conv3, conv4, conv5, conv6,
           fc_g_w, fc_g_b, fc_l_w, fc_l_b,
           gat1_W, gat1_a1, gat1_a2, gat2_W, gat2_a1, gat2_a2):
    w1 = conv1.transpose(2, 3, 1, 0).reshape(25, 32)
    w2 = conv2.transpose(2, 3, 1, 0).reshape(288, 32)
    w3 = conv3.transpose(2, 3, 1, 0).reshape(288, 64)
    w4 = conv4.transpose(2, 3, 1, 0).reshape(576, 64)
    w5 = conv5.transpose(2, 3, 1, 0).reshape(576, 128)
    w6 = conv6.transpose(2, 3, 1, 0).reshape(1152, 128)

    g = _run_global(x, w1, w2, w3, w4, w5, w6)
    wgp = fc_g_w.reshape(256, 128, 16, 16).transpose(2, 3, 1, 0).reshape(32768, 256)
    g_fc = _run_fcg(g.reshape(_B, 256 * 128), wgp, fc_g_b.reshape(1, 256))

    fcwp = fc_l_w.reshape(256, 128, 3, 3).transpose(2, 3, 1, 0)
    lfc4 = _run_local(x, px, w1, w2, w3, w4, w5, w6, fcwp, fc_l_b.reshape(1, 256))
    lfc = lfc4.reshape(_B, _N, 256)

    return _run_gat(lfc, g_fc.reshape(_B, 1, 256),
                    gat1_W, gat1_a1, gat1_a2, gat2_W, gat2_a1, gat2_a2)
